# baseline (speedup 1.0000x reference)
import functools

import jax
import jax.numpy as jnp
from jax import lax
from jax.experimental import pallas as pl
from jax.experimental.pallas import tpu as pltpu
from jax.experimental.pallas import tpu_sc as plsc

N = 10000
E = 320000
D = 128
H = 64

B_N = 1000
B_E = 2000

NC = 2
NS = 16
EH = E // 2
EPT_B = EH // (NC * NS)
KB = 40
EPT_D = EH // NS
KD = 40
RPT = 632
RPT_LAST = N - 15 * RPT

_f32 = jnp.float32


def _stage_a_body(x_ref, pos_ref, win_ref, bin_ref, wlin_ref, wsrc_ref,
                  wdst_ref, wa1_ref, drec_ref, srec_ref):
    x = x_ref[...]
    h = jnp.maximum(jnp.dot(x, win_ref[...].T, preferred_element_type=_f32)
                    + bin_ref[...], 0.0)
    wda = jnp.dot(wa1_ref[...], wdst_ref[...], preferred_element_type=_f32)
    wsa = jnp.dot(wa1_ref[...], wsrc_ref[...], preferred_element_type=_f32)
    p = jnp.dot(h, wda.T, preferred_element_type=_f32)
    q = jnp.dot(h, wsa.T, preferred_element_type=_f32)
    v = jnp.dot(h, wlin_ref[...].T, preferred_element_type=_f32)
    pospad = jnp.concatenate(
        [pos_ref[...], jnp.zeros((B_N, H - 3), _f32)], axis=1)
    drec_ref[...] = jnp.concatenate([p, pospad], axis=1)
    srec_ref[...] = jnp.concatenate([-q, -pospad, v], axis=1)


def _stage_a(x, pos, W_in, b_in2, W_lin, W_src, W_dst, Wa1):
    nblk = N // B_N
    full = pl.BlockSpec((D, D), lambda i: (0, 0))
    fullH = pl.BlockSpec((H, D), lambda i: (0, 0))
    bias = pl.BlockSpec((1, D), lambda i: (0, 0))
    return pl.pallas_call(
        _stage_a_body,
        grid=(nblk,),
        in_specs=[
            pl.BlockSpec((B_N, D), lambda i: (i, 0)),
            pl.BlockSpec((B_N, 3), lambda i: (i, 0)),
            full, bias, full, full, full, fullH,
        ],
        out_specs=[
            pl.BlockSpec((B_N, D), lambda i: (i, 0)),
            pl.BlockSpec((B_N, 2 * D), lambda i: (i, 0)),
        ],
        out_shape=[
            jax.ShapeDtypeStruct((N, D), _f32),
            jax.ShapeDtypeStruct((N, 2 * D), _f32),
        ],
    )(x, pos, W_in, b_in2, W_lin, W_src, W_dst, Wa1)


def _gather_body(drec, srec, src_h, dst_h, out,
                 idxs0, idxd0, bufd0, bufs0,
                 idxs1, idxd1, bufd1, bufs1,
                 semg0, semg1, semw0, semw1):
    c = lax.axis_index("c")
    s = lax.axis_index("s")
    wid = s * NC + c
    base = wid * EPT_B
    nchunks = EPT_B // KB

    slots = ((idxs0, idxd0, bufd0, bufs0, semg0, semw0),
             (idxs1, idxd1, bufd1, bufs1, semg1, semw1))

    def issue_gathers(slot, ci):
        idxs, idxd, bufd, bufs, semg, _ = slots[slot]
        e0 = base + ci * KB
        pltpu.sync_copy(dst_h.at[pl.ds(e0, KB)], idxd)
        pltpu.sync_copy(src_h.at[pl.ds(e0, KB)], idxs)
        pltpu.async_copy(drec.at[idxd], bufd, semg)
        pltpu.async_copy(srec.at[idxs], bufs, semg)

    def wait_gathers(slot):
        idxs, idxd, bufd, bufs, semg, _ = slots[slot]
        pltpu.make_async_copy(drec.at[idxd], bufd, semg).wait()
        pltpu.make_async_copy(srec.at[idxs], bufs, semg).wait()

    def add_rows(slot):
        _, _, bufd, bufs, _, _ = slots[slot]

        def addrow(r, carry2):
            for cc in range(D // 16):
                sl = pl.ds(cc * 16, 16)
                bufs[r, sl] = bufs[r, sl] + bufd[r, sl]
            return carry2

        lax.fori_loop(0, KB, addrow, 0)

    def issue_write(slot, ci):
        _, _, _, bufs, _, semw = slots[slot]
        e0 = base + ci * KB
        pltpu.async_copy(bufs, out.at[pl.ds(e0, KB)], semw)

    def wait_write(slot):
        _, _, _, bufs, _, semw = slots[slot]
        pltpu.make_async_copy(bufs, out.at[pl.ds(base, KB)], semw).wait()

    issue_gathers(0, 0)

    def pair(i, carry):
        c0 = 2 * i
        c1 = c0 + 1

        @pl.when(i > 0)
        def _():
            wait_write(1)

        issue_gathers(1, c1)
        wait_gathers(0)
        add_rows(0)
        issue_write(0, c0)
        wait_write(0)

        @pl.when(c0 + 2 < nchunks)
        def _():
            issue_gathers(0, c0 + 2)

        wait_gathers(1)
        add_rows(1)
        issue_write(1, c1)
        return carry

    lax.fori_loop(0, nchunks // 2, pair, 0)
    wait_write(1)
    wait_gathers(0)
    add_rows(0)
    pltpu.sync_copy(bufs0, out.at[pl.ds(base + (nchunks - 1) * KB, KB)])


_gather_kernel = functools.partial(
    pl.kernel,
    mesh=plsc.VectorSubcoreMesh(core_axis_name="c", subcore_axis_name="s"),
    out_type=jax.ShapeDtypeStruct((EH, 2 * D), _f32),
    scratch_types=[
        pltpu.VMEM((KB,), jnp.int32),
        pltpu.VMEM((KB,), jnp.int32),
        pltpu.VMEM((KB, D), _f32),
        pltpu.VMEM((KB, 2 * D), _f32),
        pltpu.VMEM((KB,), jnp.int32),
        pltpu.VMEM((KB,), jnp.int32),
        pltpu.VMEM((KB, D), _f32),
        pltpu.VMEM((KB, 2 * D), _f32),
        pltpu.SemaphoreType.DMA,
        pltpu.SemaphoreType.DMA,
        pltpu.SemaphoreType.DMA,
        pltpu.SemaphoreType.DMA,
    ],
)(_gather_body)


def _stage_c_body(g_ref, wp1p_ref, bp1_ref, wp2_ref, bp2_ref, wa1_ref,
                  ba1_ref, wa2_ref, ba2_ref, cat_ref):
    g = g_ref[...]
    u0 = g[:, 0:H]
    dp = g[:, H:2 * H]
    vv = g[:, D:2 * D]
    t = jnp.maximum(jnp.dot(dp, wp1p_ref[...].T, preferred_element_type=_f32)
                    + bp1_ref[...], 0.0)
    delta = jnp.maximum(jnp.dot(t, wp2_ref[...].T, preferred_element_type=_f32)
                        + bp2_ref[...], 0.0)
    u = jnp.maximum(u0 + jnp.dot(delta, wa1_ref[...].T,
                                 preferred_element_type=_f32)
                    + ba1_ref[...], 0.0)
    alpha = jnp.maximum(jnp.dot(u, wa2_ref[...].T, preferred_element_type=_f32)
                        + ba2_ref[...], 0.0)
    ex = jnp.exp(alpha)
    exm = ex * (vv + delta)
    cat_ref[0] = jnp.concatenate([exm[:, 0:H], ex[:, 0:H]], axis=1)
    cat_ref[1] = jnp.concatenate([exm[:, H:D], ex[:, H:D]], axis=1)


def _stage_c(g, Wp1p, bp1_2, Wp2, bp2_2, Wa1, ba1_2, Wa2, ba2_2):
    nblk = EH // B_E
    wHH = pl.BlockSpec((H, H), lambda i: (0, 0))
    wDH = pl.BlockSpec((D, H), lambda i: (0, 0))
    wHD = pl.BlockSpec((H, D), lambda i: (0, 0))
    bH = pl.BlockSpec((1, H), lambda i: (0, 0))
    bD = pl.BlockSpec((1, D), lambda i: (0, 0))
    return pl.pallas_call(
        _stage_c_body,
        grid=(nblk,),
        in_specs=[
            pl.BlockSpec((B_E, 2 * D), lambda i: (i, 0)),
            wHH, bH, wDH, bD, wHD, bH, wDH, bD,
        ],
        out_specs=pl.BlockSpec((2, B_E, D), lambda i: (0, i, 0)),
        out_shape=jax.ShapeDtypeStruct((2, EH, D), _f32),
    )(g, Wp1p, bp1_2, Wp2, bp2_2, Wa1, ba1_2, Wa2, ba2_2)


def _scatter_body(cat_h, dst_h, zeros_h, acc_out, idxd0, catb0, idxd1, catb1,
                  acc, seml0, seml1):
    c = lax.axis_index("c")
    s = lax.axis_index("s")
    r0 = s * RPT

    @pl.when(s < NS - 1)
    def _():
        pltpu.sync_copy(zeros_h.at[pl.ds(r0, RPT)], acc.at[pl.ds(r0, RPT)])

    @pl.when(s == NS - 1)
    def _():
        pltpu.sync_copy(zeros_h.at[pl.ds(r0, RPT_LAST)],
                        acc.at[pl.ds(r0, RPT_LAST)])

    plsc.subcore_barrier()

    nchunks = EPT_D // KD
    slots = ((idxd0, catb0, seml0), (idxd1, catb1, seml1))

    def issue_load(slot, ci):
        idxd, catb, seml = slots[slot]
        e0 = s * EPT_D + ci * KD
        pltpu.sync_copy(dst_h.at[pl.ds(e0, KD)], idxd)
        pltpu.async_copy(cat_h.at[c, pl.ds(e0, KD)], catb, seml)

    def wait_load(slot):
        idxd, catb, seml = slots[slot]
        pltpu.make_async_copy(cat_h.at[c, pl.ds(0, KD)], catb, seml).wait()

    def scatter(slot):
        idxd, catb, _ = slots[slot]
        pltpu.sync_copy(catb, acc.at[idxd], add=True)

    issue_load(0, 0)

    def pair(i, carry):
        c0 = 2 * i
        issue_load(1, c0 + 1)
        wait_load(0)
        scatter(0)

        @pl.when(c0 + 2 < nchunks)
        def _():
            issue_load(0, c0 + 2)

        wait_load(1)
        scatter(1)
        return carry

    lax.fori_loop(0, nchunks // 2, pair, 0)
    plsc.subcore_barrier()

    @pl.when(s < NS - 1)
    def _():
        pltpu.sync_copy(acc.at[pl.ds(r0, RPT)], acc_out.at[c, pl.ds(r0, RPT)])

    @pl.when(s == NS - 1)
    def _():
        pltpu.sync_copy(acc.at[pl.ds(r0, RPT_LAST)],
                        acc_out.at[c, pl.ds(r0, RPT_LAST)])


_scatter_kernel = functools.partial(
    pl.kernel,
    mesh=plsc.VectorSubcoreMesh(core_axis_name="c", subcore_axis_name="s"),
    out_type=jax.ShapeDtypeStruct((NC, N, D), _f32),
    scratch_types=[
        pltpu.VMEM((KD,), jnp.int32),
        pltpu.VMEM((KD, D), _f32),
        pltpu.VMEM((KD,), jnp.int32),
        pltpu.VMEM((KD, D), _f32),
        pltpu.VMEM_SHARED((N, D), _f32),
        pltpu.SemaphoreType.DMA,
        pltpu.SemaphoreType.DMA,
    ],
)(_scatter_body)


def _stage_e_body(al_ref, ar_ref, bl_ref, br_ref, wout_ref, bout_ref, o_ref):
    al = al_ref[0] + bl_ref[0]
    ar = ar_ref[0] + br_ref[0]
    rl = al[:, 0:H] / (al[:, H:D] + 1e-16)
    rr = ar[:, 0:H] / (ar[:, H:D] + 1e-16)
    w = wout_ref[...]
    o = (jnp.dot(rl, w[:, 0:H].T, preferred_element_type=_f32)
         + jnp.dot(rr, w[:, H:D].T, preferred_element_type=_f32)
         + bout_ref[...])
    o_ref[...] = jnp.maximum(o, 0.0)


def _stage_e(acc_a, acc_b, W_out, b_out2):
    nblk = N // B_N
    left = pl.BlockSpec((1, B_N, D), lambda i: (0, i, 0))
    right = pl.BlockSpec((1, B_N, D), lambda i: (1, i, 0))
    return pl.pallas_call(
        _stage_e_body,
        grid=(nblk,),
        in_specs=[
            left, right, left, right,
            pl.BlockSpec((D, D), lambda i: (0, 0)),
            pl.BlockSpec((1, D), lambda i: (0, 0)),
        ],
        out_specs=pl.BlockSpec((B_N, D), lambda i: (i, 0)),
        out_shape=jax.ShapeDtypeStruct((N, D), _f32),
    )(acc_a, acc_a, acc_b, acc_b, W_out, b_out2)


def kernel(x, pos, edge_index, W_in, b_in, W_lin, W_src, W_dst, Wp1, bp1,
           Wp2, bp2, Wa1, ba1, Wa2, ba2, W_out, b_out):
    src = edge_index[0]
    dst = edge_index[1]
    Wp1p = jnp.concatenate([Wp1, jnp.zeros((H, H - 3), _f32)], axis=1)
    b_in2 = b_in.reshape(1, D)
    bp1_2 = bp1.reshape(1, H)
    bp2_2 = bp2.reshape(1, D)
    ba1_2 = ba1.reshape(1, H)
    ba2_2 = ba2.reshape(1, D)
    b_out2 = b_out.reshape(1, D)

    drec, srec = _stage_a(x, pos, W_in, b_in2, W_lin, W_src, W_dst, Wa1)
    src1, src2 = src[:EH], src[EH:]
    dst1, dst2 = dst[:EH], dst[EH:]
    zeros_n = jnp.zeros((N, D), _f32)
    g1 = _gather_kernel(drec, srec, src1, dst1)
    g2 = _gather_kernel(drec, srec, src2, dst2)
    cat1 = _stage_c(g1, Wp1p, bp1_2, Wp2, bp2_2, Wa1, ba1_2, Wa2, ba2_2)
    cat2 = _stage_c(g2, Wp1p, bp1_2, Wp2, bp2_2, Wa1, ba1_2, Wa2, ba2_2)
    acc_a = _scatter_kernel(cat1, dst1, zeros_n)
    acc_b = _scatter_kernel(cat2, dst2, zeros_n)
    return _stage_e(acc_a, acc_b, W_out, b_out2)


# speedup vs baseline: 1.2272x; 1.2272x over previous
import functools

import jax
import jax.numpy as jnp
from jax import lax
from jax.experimental import pallas as pl
from jax.experimental.pallas import tpu as pltpu
from jax.experimental.pallas import tpu_sc as plsc

N = 10000
E = 320000
D = 128
H = 64

B_N = 1000

NC = 2
NS = 16
E1 = 163840
E2 = E - E1
KB = 80
KD = 80
B_E = 2560
RPT = 632
RPT_LAST = N - 15 * RPT

_f32 = jnp.float32


def _stage_a_body(x_ref, pos_ref, win_ref, bin_ref, wlin_ref, wsrc_ref,
                  wdst_ref, wa1_ref, drec_ref, srec_ref):
    x = x_ref[...]
    h = jnp.maximum(jnp.dot(x, win_ref[...].T, preferred_element_type=_f32)
                    + bin_ref[...], 0.0)
    wda = jnp.dot(wa1_ref[...], wdst_ref[...], preferred_element_type=_f32)
    wsa = jnp.dot(wa1_ref[...], wsrc_ref[...], preferred_element_type=_f32)
    p = jnp.dot(h, wda.T, preferred_element_type=_f32)
    q = jnp.dot(h, wsa.T, preferred_element_type=_f32)
    v = jnp.dot(h, wlin_ref[...].T, preferred_element_type=_f32)
    pospad = jnp.concatenate(
        [pos_ref[...], jnp.zeros((B_N, H - 3), _f32)], axis=1)
    drec_ref[...] = jnp.concatenate([p, pospad], axis=1)
    srec_ref[...] = jnp.concatenate([-q, -pospad, v], axis=1)


def _stage_a(x, pos, W_in, b_in2, W_lin, W_src, W_dst, Wa1):
    nblk = N // B_N
    full = pl.BlockSpec((D, D), lambda i: (0, 0))
    fullH = pl.BlockSpec((H, D), lambda i: (0, 0))
    bias = pl.BlockSpec((1, D), lambda i: (0, 0))
    return pl.pallas_call(
        _stage_a_body,
        grid=(nblk,),
        in_specs=[
            pl.BlockSpec((B_N, D), lambda i: (i, 0)),
            pl.BlockSpec((B_N, 3), lambda i: (i, 0)),
            full, bias, full, full, full, fullH,
        ],
        out_specs=[
            pl.BlockSpec((B_N, D), lambda i: (i, 0)),
            pl.BlockSpec((B_N, 2 * D), lambda i: (i, 0)),
        ],
        out_shape=[
            jax.ShapeDtypeStruct((N, D), _f32),
            jax.ShapeDtypeStruct((N, 2 * D), _f32),
        ],
    )(x, pos, W_in, b_in2, W_lin, W_src, W_dst, Wa1)


def _make_gather_body(ept):
    nchunks_s = ept // KB

    def _gather_body(drec, srec, src_h, dst_h, out,
                     idxs0, idxd0, bufd0, bufs0,
                     idxs1, idxd1, bufd1, bufs1,
                     semg0, semg1, semw0, semw1):
        c = lax.axis_index("c")
        s = lax.axis_index("s")
        wid = s * NC + c
        base = wid * ept
        nchunks = nchunks_s

        slots = ((idxs0, idxd0, bufd0, bufs0, semg0, semw0),
                 (idxs1, idxd1, bufd1, bufs1, semg1, semw1))

        def issue_gathers(slot, ci):
            idxs, idxd, bufd, bufs, semg, _ = slots[slot]
            e0 = base + ci * KB
            pltpu.sync_copy(dst_h.at[pl.ds(e0, KB)], idxd)
            pltpu.sync_copy(src_h.at[pl.ds(e0, KB)], idxs)
            pltpu.async_copy(drec.at[idxd], bufd, semg)
            pltpu.async_copy(srec.at[idxs], bufs, semg)

        def wait_gathers(slot):
            idxs, idxd, bufd, bufs, semg, _ = slots[slot]
            pltpu.make_async_copy(drec.at[idxd], bufd, semg).wait()
            pltpu.make_async_copy(srec.at[idxs], bufs, semg).wait()

        def add_rows(slot):
            _, _, bufd, bufs, _, _ = slots[slot]

            def addrow(r, carry2):
                for cc in range(D // 16):
                    sl = pl.ds(cc * 16, 16)
                    bufs[r, sl] = bufs[r, sl] + bufd[r, sl]
                return carry2

            lax.fori_loop(0, KB, addrow, 0)

        def issue_write(slot, ci):
            _, _, _, bufs, _, semw = slots[slot]
            e0 = base + ci * KB
            pltpu.async_copy(bufs, out.at[pl.ds(e0, KB)], semw)

        def wait_write(slot):
            _, _, _, bufs, _, semw = slots[slot]
            pltpu.make_async_copy(bufs, out.at[pl.ds(base, KB)], semw).wait()

        issue_gathers(0, 0)

        def pair(i, carry):
            c0 = 2 * i
            c1 = c0 + 1

            @pl.when(i > 0)
            def _():
                wait_write(1)

            issue_gathers(1, c1)
            wait_gathers(0)
            add_rows(0)
            issue_write(0, c0)
            wait_write(0)

            @pl.when(c0 + 2 < nchunks)
            def _():
                issue_gathers(0, c0 + 2)

            wait_gathers(1)
            add_rows(1)
            issue_write(1, c1)
            return carry

        lax.fori_loop(0, nchunks // 2, pair, 0)
        wait_write(1)
        if nchunks % 2 == 1:
            wait_gathers(0)
            add_rows(0)
            pltpu.sync_copy(bufs0,
                            out.at[pl.ds(base + (nchunks - 1) * KB, KB)])

    return _gather_body


def _make_gather_kernel(eh):
    return functools.partial(
        pl.kernel,
        mesh=plsc.VectorSubcoreMesh(core_axis_name="c", subcore_axis_name="s"),
        out_type=jax.ShapeDtypeStruct((eh, 2 * D), _f32),
        scratch_types=[
            pltpu.VMEM((KB,), jnp.int32),
            pltpu.VMEM((KB,), jnp.int32),
            pltpu.VMEM((KB, D), _f32),
            pltpu.VMEM((KB, 2 * D), _f32),
            pltpu.VMEM((KB,), jnp.int32),
            pltpu.VMEM((KB,), jnp.int32),
            pltpu.VMEM((KB, D), _f32),
            pltpu.VMEM((KB, 2 * D), _f32),
            pltpu.SemaphoreType.DMA,
            pltpu.SemaphoreType.DMA,
            pltpu.SemaphoreType.DMA,
            pltpu.SemaphoreType.DMA,
        ],
    )(_make_gather_body(eh // (NC * NS)))


_gather_kernel_1 = _make_gather_kernel(E1)
_gather_kernel_2 = _make_gather_kernel(E2)


def _stage_c_body(g_ref, wp1p_ref, bp1_ref, wp2_ref, bp2_ref, wa1_ref,
                  ba1_ref, wa2_ref, ba2_ref, cat_ref):
    g = g_ref[...]
    u0 = g[:, 0:H]
    dp = g[:, H:2 * H]
    vv = g[:, D:2 * D]
    t = jnp.maximum(jnp.dot(dp, wp1p_ref[...].T, preferred_element_type=_f32)
                    + bp1_ref[...], 0.0)
    delta = jnp.maximum(jnp.dot(t, wp2_ref[...].T, preferred_element_type=_f32)
                        + bp2_ref[...], 0.0)
    u = jnp.maximum(u0 + jnp.dot(delta, wa1_ref[...].T,
                                 preferred_element_type=_f32)
                    + ba1_ref[...], 0.0)
    alpha = jnp.maximum(jnp.dot(u, wa2_ref[...].T, preferred_element_type=_f32)
                        + ba2_ref[...], 0.0)
    ex = jnp.exp(alpha)
    exm = ex * (vv + delta)
    cat_ref[0] = jnp.concatenate([exm[:, 0:H], ex[:, 0:H]], axis=1)
    cat_ref[1] = jnp.concatenate([exm[:, H:D], ex[:, H:D]], axis=1)


def _stage_c(g, Wp1p, bp1_2, Wp2, bp2_2, Wa1, ba1_2, Wa2, ba2_2):
    eh = g.shape[0]
    nblk = eh // B_E
    wHH = pl.BlockSpec((H, H), lambda i: (0, 0))
    wDH = pl.BlockSpec((D, H), lambda i: (0, 0))
    wHD = pl.BlockSpec((H, D), lambda i: (0, 0))
    bH = pl.BlockSpec((1, H), lambda i: (0, 0))
    bD = pl.BlockSpec((1, D), lambda i: (0, 0))
    return pl.pallas_call(
        _stage_c_body,
        grid=(nblk,),
        in_specs=[
            pl.BlockSpec((B_E, 2 * D), lambda i: (i, 0)),
            wHH, bH, wDH, bD, wHD, bH, wDH, bD,
        ],
        out_specs=pl.BlockSpec((2, B_E, D), lambda i: (0, i, 0)),
        out_shape=jax.ShapeDtypeStruct((2, eh, D), _f32),
    )(g, Wp1p, bp1_2, Wp2, bp2_2, Wa1, ba1_2, Wa2, ba2_2)


def _make_scatter_body(ept):
    nchunks_s = ept // KD

    def _scatter_body(cat_h, dst_h, zeros_h, acc_out, idxd0, catb0, idxd1,
                      catb1, acc, seml0, seml1):
        c = lax.axis_index("c")
        s = lax.axis_index("s")
        r0 = s * RPT

        @pl.when(s < NS - 1)
        def _():
            pltpu.sync_copy(zeros_h.at[pl.ds(r0, RPT)],
                            acc.at[pl.ds(r0, RPT)])

        @pl.when(s == NS - 1)
        def _():
            pltpu.sync_copy(zeros_h.at[pl.ds(r0, RPT_LAST)],
                            acc.at[pl.ds(r0, RPT_LAST)])

        plsc.subcore_barrier()

        nchunks = nchunks_s
        slots = ((idxd0, catb0, seml0), (idxd1, catb1, seml1))

        def issue_load(slot, ci):
            idxd, catb, seml = slots[slot]
            e0 = s * ept + ci * KD
            pltpu.sync_copy(dst_h.at[pl.ds(e0, KD)], idxd)
            pltpu.async_copy(cat_h.at[c, pl.ds(e0, KD)], catb, seml)

        def wait_load(slot):
            idxd, catb, seml = slots[slot]
            pltpu.make_async_copy(cat_h.at[c, pl.ds(0, KD)], catb,
                                  seml).wait()

        def scatter(slot):
            idxd, catb, _ = slots[slot]
            pltpu.sync_copy(catb, acc.at[idxd], add=True)

        issue_load(0, 0)

        def pair(i, carry):
            c0 = 2 * i
            issue_load(1, c0 + 1)
            wait_load(0)
            scatter(0)

            @pl.when(c0 + 2 < nchunks)
            def _():
                issue_load(0, c0 + 2)

            wait_load(1)
            scatter(1)
            return carry

        lax.fori_loop(0, nchunks // 2, pair, 0)
        plsc.subcore_barrier()

        @pl.when(s < NS - 1)
        def _():
            pltpu.sync_copy(acc.at[pl.ds(r0, RPT)],
                            acc_out.at[c, pl.ds(r0, RPT)])

        @pl.when(s == NS - 1)
        def _():
            pltpu.sync_copy(acc.at[pl.ds(r0, RPT_LAST)],
                            acc_out.at[c, pl.ds(r0, RPT_LAST)])

    return _scatter_body


def _make_scatter_kernel(eh):
    return functools.partial(
        pl.kernel,
        mesh=plsc.VectorSubcoreMesh(core_axis_name="c", subcore_axis_name="s"),
        out_type=jax.ShapeDtypeStruct((NC, N, D), _f32),
        scratch_types=[
            pltpu.VMEM((KD,), jnp.int32),
            pltpu.VMEM((KD, D), _f32),
            pltpu.VMEM((KD,), jnp.int32),
            pltpu.VMEM((KD, D), _f32),
            pltpu.VMEM_SHARED((N, D), _f32),
            pltpu.SemaphoreType.DMA,
            pltpu.SemaphoreType.DMA,
        ],
    )(_make_scatter_body(eh // NS))


_scatter_kernel_1 = _make_scatter_kernel(E1)
_scatter_kernel_2 = _make_scatter_kernel(E2)


def _stage_e_body(al_ref, ar_ref, bl_ref, br_ref, wout_ref, bout_ref, o_ref):
    al = al_ref[0] + bl_ref[0]
    ar = ar_ref[0] + br_ref[0]
    rl = al[:, 0:H] / (al[:, H:D] + 1e-16)
    rr = ar[:, 0:H] / (ar[:, H:D] + 1e-16)
    w = wout_ref[...]
    o = (jnp.dot(rl, w[:, 0:H].T, preferred_element_type=_f32)
         + jnp.dot(rr, w[:, H:D].T, preferred_element_type=_f32)
         + bout_ref[...])
    o_ref[...] = jnp.maximum(o, 0.0)


def _stage_e(acc_a, acc_b, W_out, b_out2):
    nblk = N // B_N
    left = pl.BlockSpec((1, B_N, D), lambda i: (0, i, 0))
    right = pl.BlockSpec((1, B_N, D), lambda i: (1, i, 0))
    return pl.pallas_call(
        _stage_e_body,
        grid=(nblk,),
        in_specs=[
            left, right, left, right,
            pl.BlockSpec((D, D), lambda i: (0, 0)),
            pl.BlockSpec((1, D), lambda i: (0, 0)),
        ],
        out_specs=pl.BlockSpec((B_N, D), lambda i: (i, 0)),
        out_shape=jax.ShapeDtypeStruct((N, D), _f32),
    )(acc_a, acc_a, acc_b, acc_b, W_out, b_out2)


def kernel(x, pos, edge_index, W_in, b_in, W_lin, W_src, W_dst, Wp1, bp1,
           Wp2, bp2, Wa1, ba1, Wa2, ba2, W_out, b_out):
    src = edge_index[0]
    dst = edge_index[1]
    Wp1p = jnp.concatenate([Wp1, jnp.zeros((H, H - 3), _f32)], axis=1)
    b_in2 = b_in.reshape(1, D)
    bp1_2 = bp1.reshape(1, H)
    bp2_2 = bp2.reshape(1, D)
    ba1_2 = ba1.reshape(1, H)
    ba2_2 = ba2.reshape(1, D)
    b_out2 = b_out.reshape(1, D)

    drec, srec = _stage_a(x, pos, W_in, b_in2, W_lin, W_src, W_dst, Wa1)
    src1, src2 = src[:E1], src[E1:]
    dst1, dst2 = dst[:E1], dst[E1:]
    zeros_n = jnp.zeros((N, D), _f32)
    g1 = _gather_kernel_1(drec, srec, src1, dst1)
    g2 = _gather_kernel_2(drec, srec, src2, dst2)
    cat1 = _stage_c(g1, Wp1p, bp1_2, Wp2, bp2_2, Wa1, ba1_2, Wa2, ba2_2)
    cat2 = _stage_c(g2, Wp1p, bp1_2, Wp2, bp2_2, Wa1, ba1_2, Wa2, ba2_2)
    acc_a = _scatter_kernel_1(cat1, dst1, zeros_n)
    acc_b = _scatter_kernel_2(cat2, dst2, zeros_n)
    return _stage_e(acc_a, acc_b, W_out, b_out2)


# speedup vs baseline: 1.2969x; 1.0567x over previous
import functools

import jax
import jax.numpy as jnp
from jax import lax
from jax.experimental import pallas as pl
from jax.experimental.pallas import tpu as pltpu
from jax.experimental.pallas import tpu_sc as plsc

N = 10000
E = 320000
D = 128
H = 64

B_N = 1000

NC = 2
NS = 16
E1 = 163840
E2 = E - E1
KB = 80
KD = 160
B_E = 2560
RPT = 632
RPT_LAST = N - 15 * RPT

_f32 = jnp.float32


def _stage_a_body(x_ref, pos_ref, win_ref, bin_ref, wlin_ref, wsrc_ref,
                  wdst_ref, wa1_ref, drec_ref, srec_ref):
    x = x_ref[...]
    h = jnp.maximum(jnp.dot(x, win_ref[...].T, preferred_element_type=_f32)
                    + bin_ref[...], 0.0)
    wda = jnp.dot(wa1_ref[...], wdst_ref[...], preferred_element_type=_f32)
    wsa = jnp.dot(wa1_ref[...], wsrc_ref[...], preferred_element_type=_f32)
    p = jnp.dot(h, wda.T, preferred_element_type=_f32)
    q = jnp.dot(h, wsa.T, preferred_element_type=_f32)
    v = jnp.dot(h, wlin_ref[...].T, preferred_element_type=_f32)
    pospad = jnp.concatenate(
        [pos_ref[...], jnp.zeros((B_N, H - 3), _f32)], axis=1)
    drec_ref[...] = jnp.concatenate([p, pospad], axis=1)
    srec_ref[...] = jnp.concatenate([-q, -pospad, v], axis=1)


def _stage_a(x, pos, W_in, b_in2, W_lin, W_src, W_dst, Wa1):
    nblk = N // B_N
    full = pl.BlockSpec((D, D), lambda i: (0, 0))
    fullH = pl.BlockSpec((H, D), lambda i: (0, 0))
    bias = pl.BlockSpec((1, D), lambda i: (0, 0))
    return pl.pallas_call(
        _stage_a_body,
        grid=(nblk,),
        in_specs=[
            pl.BlockSpec((B_N, D), lambda i: (i, 0)),
            pl.BlockSpec((B_N, 3), lambda i: (i, 0)),
            full, bias, full, full, full, fullH,
        ],
        out_specs=[
            pl.BlockSpec((B_N, D), lambda i: (i, 0)),
            pl.BlockSpec((B_N, 2 * D), lambda i: (i, 0)),
        ],
        out_shape=[
            jax.ShapeDtypeStruct((N, D), _f32),
            jax.ShapeDtypeStruct((N, 2 * D), _f32),
        ],
    )(x, pos, W_in, b_in2, W_lin, W_src, W_dst, Wa1)


def _make_gather_body(ept):
    nchunks_s = ept // KB

    def _gather_body(drec, srec, src_h, dst_h, out,
                     idxs0, idxd0, bufd0, bufs0,
                     idxs1, idxd1, bufd1, bufs1,
                     semg0, semg1, semw0, semw1):
        c = lax.axis_index("c")
        s = lax.axis_index("s")
        wid = s * NC + c
        base = wid * ept
        nchunks = nchunks_s

        slots = ((idxs0, idxd0, bufd0, bufs0, semg0, semw0),
                 (idxs1, idxd1, bufd1, bufs1, semg1, semw1))

        def issue_gathers(slot, ci):
            idxs, idxd, bufd, bufs, semg, _ = slots[slot]
            e0 = base + ci * KB
            pltpu.sync_copy(dst_h.at[pl.ds(e0, KB)], idxd)
            pltpu.sync_copy(src_h.at[pl.ds(e0, KB)], idxs)
            pltpu.async_copy(drec.at[idxd], bufd, semg)
            pltpu.async_copy(srec.at[idxs], bufs, semg)

        def wait_gathers(slot):
            idxs, idxd, bufd, bufs, semg, _ = slots[slot]
            pltpu.make_async_copy(drec.at[idxd], bufd, semg).wait()
            pltpu.make_async_copy(srec.at[idxs], bufs, semg).wait()

        def add_rows(slot):
            _, _, bufd, bufs, _, _ = slots[slot]

            def addrow(r, carry2):
                for cc in range(D // 16):
                    sl = pl.ds(cc * 16, 16)
                    bufs[r, sl] = bufs[r, sl] + bufd[r, sl]
                return carry2

            lax.fori_loop(0, KB, addrow, 0)

        def issue_write(slot, ci):
            _, _, _, bufs, _, semw = slots[slot]
            e0 = base + ci * KB
            pltpu.async_copy(bufs, out.at[pl.ds(e0, KB)], semw)

        def wait_write(slot):
            _, _, _, bufs, _, semw = slots[slot]
            pltpu.make_async_copy(bufs, out.at[pl.ds(base, KB)], semw).wait()

        issue_gathers(0, 0)

        def pair(i, carry):
            c0 = 2 * i
            c1 = c0 + 1

            @pl.when(i > 0)
            def _():
                wait_write(1)

            issue_gathers(1, c1)
            wait_gathers(0)
            add_rows(0)
            issue_write(0, c0)
            wait_write(0)

            @pl.when(c0 + 2 < nchunks)
            def _():
                issue_gathers(0, c0 + 2)

            wait_gathers(1)
            add_rows(1)
            issue_write(1, c1)
            return carry

        lax.fori_loop(0, nchunks // 2, pair, 0)
        wait_write(1)
        if nchunks % 2 == 1:
            wait_gathers(0)
            add_rows(0)
            pltpu.sync_copy(bufs0,
                            out.at[pl.ds(base + (nchunks - 1) * KB, KB)])

    return _gather_body


def _make_gather_kernel(eh):
    return functools.partial(
        pl.kernel,
        mesh=plsc.VectorSubcoreMesh(core_axis_name="c", subcore_axis_name="s"),
        out_type=jax.ShapeDtypeStruct((eh, 2 * D), _f32),
        scratch_types=[
            pltpu.VMEM((KB,), jnp.int32),
            pltpu.VMEM((KB,), jnp.int32),
            pltpu.VMEM((KB, D), _f32),
            pltpu.VMEM((KB, 2 * D), _f32),
            pltpu.VMEM((KB,), jnp.int32),
            pltpu.VMEM((KB,), jnp.int32),
            pltpu.VMEM((KB, D), _f32),
            pltpu.VMEM((KB, 2 * D), _f32),
            pltpu.SemaphoreType.DMA,
            pltpu.SemaphoreType.DMA,
            pltpu.SemaphoreType.DMA,
            pltpu.SemaphoreType.DMA,
        ],
    )(_make_gather_body(eh // (NC * NS)))


_gather_kernel_1 = _make_gather_kernel(E1)
_gather_kernel_2 = _make_gather_kernel(E2)


def _stage_c_body(g_ref, wp1p_ref, bp1_ref, wp2_ref, bp2_ref, wa1_ref,
                  ba1_ref, wa2_ref, ba2_ref, cat_ref):
    g = g_ref[...]
    u0 = g[:, 0:H]
    dp = g[:, H:2 * H]
    vv = g[:, D:2 * D]
    t = jnp.maximum(jnp.dot(dp, wp1p_ref[...].T, preferred_element_type=_f32)
                    + bp1_ref[...], 0.0)
    delta = jnp.maximum(jnp.dot(t, wp2_ref[...].T, preferred_element_type=_f32)
                        + bp2_ref[...], 0.0)
    u = jnp.maximum(u0 + jnp.dot(delta, wa1_ref[...].T,
                                 preferred_element_type=_f32)
                    + ba1_ref[...], 0.0)
    alpha = jnp.maximum(jnp.dot(u, wa2_ref[...].T, preferred_element_type=_f32)
                        + ba2_ref[...], 0.0)
    ex = jnp.exp(alpha)
    exm = ex * (vv + delta)
    cat_ref[0] = jnp.concatenate([exm[:, 0:H], ex[:, 0:H]], axis=1)
    cat_ref[1] = jnp.concatenate([exm[:, H:D], ex[:, H:D]], axis=1)


def _stage_c(g, Wp1p, bp1_2, Wp2, bp2_2, Wa1, ba1_2, Wa2, ba2_2):
    eh = g.shape[0]
    nblk = eh // B_E
    wHH = pl.BlockSpec((H, H), lambda i: (0, 0))
    wDH = pl.BlockSpec((D, H), lambda i: (0, 0))
    wHD = pl.BlockSpec((H, D), lambda i: (0, 0))
    bH = pl.BlockSpec((1, H), lambda i: (0, 0))
    bD = pl.BlockSpec((1, D), lambda i: (0, 0))
    return pl.pallas_call(
        _stage_c_body,
        grid=(nblk,),
        in_specs=[
            pl.BlockSpec((B_E, 2 * D), lambda i: (i, 0)),
            wHH, bH, wDH, bD, wHD, bH, wDH, bD,
        ],
        out_specs=pl.BlockSpec((2, B_E, D), lambda i: (0, i, 0)),
        out_shape=jax.ShapeDtypeStruct((2, eh, D), _f32),
    )(g, Wp1p, bp1_2, Wp2, bp2_2, Wa1, ba1_2, Wa2, ba2_2)


def _make_scatter_body(ept):
    nchunks_s = ept // KD

    def _scatter_body(cat_h, dst_h, zeros_h, acc_out, idxd0, catb0, idxd1,
                      catb1, acc, seml0, seml1):
        c = lax.axis_index("c")
        s = lax.axis_index("s")
        r0 = s * RPT

        @pl.when(s < NS - 1)
        def _():
            pltpu.sync_copy(zeros_h.at[pl.ds(r0, RPT)],
                            acc.at[pl.ds(r0, RPT)])

        @pl.when(s == NS - 1)
        def _():
            pltpu.sync_copy(zeros_h.at[pl.ds(r0, RPT_LAST)],
                            acc.at[pl.ds(r0, RPT_LAST)])

        plsc.subcore_barrier()

        nchunks = nchunks_s
        slots = ((idxd0, catb0, seml0), (idxd1, catb1, seml1))

        def issue_load(slot, ci):
            idxd, catb, seml = slots[slot]
            e0 = s * ept + ci * KD
            pltpu.sync_copy(dst_h.at[pl.ds(e0, KD)], idxd)
            pltpu.async_copy(cat_h.at[c, pl.ds(e0, KD)], catb, seml)

        def wait_load(slot):
            idxd, catb, seml = slots[slot]
            pltpu.make_async_copy(cat_h.at[c, pl.ds(0, KD)], catb,
                                  seml).wait()

        def scatter(slot):
            idxd, catb, _ = slots[slot]
            pltpu.sync_copy(catb, acc.at[idxd], add=True)

        issue_load(0, 0)

        def pair(i, carry):
            c0 = 2 * i
            issue_load(1, c0 + 1)
            wait_load(0)
            scatter(0)

            @pl.when(c0 + 2 < nchunks)
            def _():
                issue_load(0, c0 + 2)

            wait_load(1)
            scatter(1)
            return carry

        lax.fori_loop(0, nchunks // 2, pair, 0)
        if nchunks % 2 == 1:
            wait_load(0)
            scatter(0)
        plsc.subcore_barrier()

        @pl.when(s < NS - 1)
        def _():
            pltpu.sync_copy(acc.at[pl.ds(r0, RPT)],
                            acc_out.at[c, pl.ds(r0, RPT)])

        @pl.when(s == NS - 1)
        def _():
            pltpu.sync_copy(acc.at[pl.ds(r0, RPT_LAST)],
                            acc_out.at[c, pl.ds(r0, RPT_LAST)])

    return _scatter_body


def _make_scatter_kernel(eh):
    return functools.partial(
        pl.kernel,
        mesh=plsc.VectorSubcoreMesh(core_axis_name="c", subcore_axis_name="s"),
        out_type=jax.ShapeDtypeStruct((NC, N, D), _f32),
        scratch_types=[
            pltpu.VMEM((KD,), jnp.int32),
            pltpu.VMEM((KD, D), _f32),
            pltpu.VMEM((KD,), jnp.int32),
            pltpu.VMEM((KD, D), _f32),
            pltpu.VMEM_SHARED((N, D), _f32),
            pltpu.SemaphoreType.DMA,
            pltpu.SemaphoreType.DMA,
        ],
    )(_make_scatter_body(eh // NS))


_scatter_kernel_1 = _make_scatter_kernel(E1)
_scatter_kernel_2 = _make_scatter_kernel(E2)


def _stage_e_body(al_ref, ar_ref, bl_ref, br_ref, wout_ref, bout_ref, o_ref):
    al = al_ref[0] + bl_ref[0]
    ar = ar_ref[0] + br_ref[0]
    rl = al[:, 0:H] / (al[:, H:D] + 1e-16)
    rr = ar[:, 0:H] / (ar[:, H:D] + 1e-16)
    w = wout_ref[...]
    o = (jnp.dot(rl, w[:, 0:H].T, preferred_element_type=_f32)
         + jnp.dot(rr, w[:, H:D].T, preferred_element_type=_f32)
         + bout_ref[...])
    o_ref[...] = jnp.maximum(o, 0.0)


def _stage_e(acc_a, acc_b, W_out, b_out2):
    nblk = N // B_N
    left = pl.BlockSpec((1, B_N, D), lambda i: (0, i, 0))
    right = pl.BlockSpec((1, B_N, D), lambda i: (1, i, 0))
    return pl.pallas_call(
        _stage_e_body,
        grid=(nblk,),
        in_specs=[
            left, right, left, right,
            pl.BlockSpec((D, D), lambda i: (0, 0)),
            pl.BlockSpec((1, D), lambda i: (0, 0)),
        ],
        out_specs=pl.BlockSpec((B_N, D), lambda i: (i, 0)),
        out_shape=jax.ShapeDtypeStruct((N, D), _f32),
    )(acc_a, acc_a, acc_b, acc_b, W_out, b_out2)


def kernel(x, pos, edge_index, W_in, b_in, W_lin, W_src, W_dst, Wp1, bp1,
           Wp2, bp2, Wa1, ba1, Wa2, ba2, W_out, b_out):
    src = edge_index[0]
    dst = edge_index[1]
    Wp1p = jnp.concatenate([Wp1, jnp.zeros((H, H - 3), _f32)], axis=1)
    b_in2 = b_in.reshape(1, D)
    bp1_2 = bp1.reshape(1, H)
    bp2_2 = bp2.reshape(1, D)
    ba1_2 = ba1.reshape(1, H)
    ba2_2 = ba2.reshape(1, D)
    b_out2 = b_out.reshape(1, D)

    drec, srec = _stage_a(x, pos, W_in, b_in2, W_lin, W_src, W_dst, Wa1)
    src1, src2 = src[:E1], src[E1:]
    dst1, dst2 = dst[:E1], dst[E1:]
    zeros_n = jnp.zeros((N, D), _f32)
    g1 = _gather_kernel_1(drec, srec, src1, dst1)
    g2 = _gather_kernel_2(drec, srec, src2, dst2)
    cat1 = _stage_c(g1, Wp1p, bp1_2, Wp2, bp2_2, Wa1, ba1_2, Wa2, ba2_2)
    cat2 = _stage_c(g2, Wp1p, bp1_2, Wp2, bp2_2, Wa1, ba1_2, Wa2, ba2_2)
    acc_a = _scatter_kernel_1(cat1, dst1, zeros_n)
    acc_b = _scatter_kernel_2(cat2, dst2, zeros_n)
    return _stage_e(acc_a, acc_b, W_out, b_out2)


# speedup vs baseline: 1.3302x; 1.0257x over previous
import functools

import jax
import jax.numpy as jnp
from jax import lax
from jax.experimental import pallas as pl
from jax.experimental.pallas import tpu as pltpu
from jax.experimental.pallas import tpu_sc as plsc

N = 10000
E = 320000
D = 128
H = 64

B_N = 1000

NC = 2
NS = 16
E1 = 163840
E2 = E - E1
KB = 80
KD = 160
B_E = 2560
RPT = 632
RPT_LAST = N - 15 * RPT

_f32 = jnp.float32


def _stage_a_body(x_ref, pos_ref, win_ref, bin_ref, wlin_ref, wsrc_ref,
                  wdst_ref, wa1_ref, drec_ref, srec_ref):
    x = x_ref[...]
    h = jnp.maximum(jnp.dot(x, win_ref[...].T, preferred_element_type=_f32)
                    + bin_ref[...], 0.0)
    wda = jnp.dot(wa1_ref[...], wdst_ref[...], preferred_element_type=_f32)
    wsa = jnp.dot(wa1_ref[...], wsrc_ref[...], preferred_element_type=_f32)
    p = jnp.dot(h, wda.T, preferred_element_type=_f32)
    q = jnp.dot(h, wsa.T, preferred_element_type=_f32)
    v = jnp.dot(h, wlin_ref[...].T, preferred_element_type=_f32)
    pospad = jnp.concatenate(
        [pos_ref[...], jnp.zeros((B_N, H - 3), _f32)], axis=1)
    drec_ref[...] = jnp.concatenate([p, pospad], axis=1)
    srec_ref[...] = jnp.concatenate([-q, -pospad, v], axis=1)


def _stage_a(x, pos, W_in, b_in2, W_lin, W_src, W_dst, Wa1):
    nblk = N // B_N
    full = pl.BlockSpec((D, D), lambda i: (0, 0))
    fullH = pl.BlockSpec((H, D), lambda i: (0, 0))
    bias = pl.BlockSpec((1, D), lambda i: (0, 0))
    return pl.pallas_call(
        _stage_a_body,
        grid=(nblk,),
        in_specs=[
            pl.BlockSpec((B_N, D), lambda i: (i, 0)),
            pl.BlockSpec((B_N, 3), lambda i: (i, 0)),
            full, bias, full, full, full, fullH,
        ],
        out_specs=[
            pl.BlockSpec((B_N, D), lambda i: (i, 0)),
            pl.BlockSpec((B_N, 2 * D), lambda i: (i, 0)),
        ],
        out_shape=[
            jax.ShapeDtypeStruct((N, D), _f32),
            jax.ShapeDtypeStruct((N, 2 * D), _f32),
        ],
    )(x, pos, W_in, b_in2, W_lin, W_src, W_dst, Wa1)


def _make_gather_body(ept, kb):
    nchunks_s = ept // kb
    KB = kb

    def _gather_body(drec, srec, src_h, dst_h, out,
                     idxs0, idxd0, bufd0, bufs0,
                     idxs1, idxd1, bufd1, bufs1,
                     semg0, semg1, semw0, semw1):
        c = lax.axis_index("c")
        s = lax.axis_index("s")
        wid = s * NC + c
        base = wid * ept
        nchunks = nchunks_s

        slots = ((idxs0, idxd0, bufd0, bufs0, semg0, semw0),
                 (idxs1, idxd1, bufd1, bufs1, semg1, semw1))

        def issue_gathers(slot, ci):
            idxs, idxd, bufd, bufs, semg, _ = slots[slot]
            e0 = base + ci * KB
            pltpu.sync_copy(dst_h.at[pl.ds(e0, KB)], idxd)
            pltpu.sync_copy(src_h.at[pl.ds(e0, KB)], idxs)
            pltpu.async_copy(drec.at[idxd], bufd, semg)
            pltpu.async_copy(srec.at[idxs], bufs, semg)

        def wait_gathers(slot):
            idxs, idxd, bufd, bufs, semg, _ = slots[slot]
            pltpu.make_async_copy(drec.at[idxd], bufd, semg).wait()
            pltpu.make_async_copy(srec.at[idxs], bufs, semg).wait()

        def add_rows(slot):
            _, _, bufd, bufs, _, _ = slots[slot]

            def addrow(r, carry2):
                for cc in range(D // 16):
                    sl = pl.ds(cc * 16, 16)
                    bufs[r, sl] = bufs[r, sl] + bufd[r, sl]
                return carry2

            lax.fori_loop(0, KB, addrow, 0)

        def issue_write(slot, ci):
            _, _, _, bufs, _, semw = slots[slot]
            e0 = base + ci * KB
            pltpu.async_copy(bufs, out.at[pl.ds(e0, KB)], semw)

        def wait_write(slot):
            _, _, _, bufs, _, semw = slots[slot]
            pltpu.make_async_copy(bufs, out.at[pl.ds(base, KB)], semw).wait()

        issue_gathers(0, 0)

        def pair(i, carry):
            c0 = 2 * i
            c1 = c0 + 1

            @pl.when(i > 0)
            def _():
                wait_write(1)

            issue_gathers(1, c1)
            wait_gathers(0)
            add_rows(0)
            issue_write(0, c0)
            wait_write(0)

            @pl.when(c0 + 2 < nchunks)
            def _():
                issue_gathers(0, c0 + 2)

            wait_gathers(1)
            add_rows(1)
            issue_write(1, c1)
            return carry

        lax.fori_loop(0, nchunks // 2, pair, 0)
        wait_write(1)
        if nchunks % 2 == 1:
            wait_gathers(0)
            add_rows(0)
            pltpu.sync_copy(bufs0,
                            out.at[pl.ds(base + (nchunks - 1) * KB, KB)])

    return _gather_body


def _make_gather_kernel(eh, kb):
    return functools.partial(
        pl.kernel,
        mesh=plsc.VectorSubcoreMesh(core_axis_name="c", subcore_axis_name="s"),
        out_type=jax.ShapeDtypeStruct((eh, 2 * D), _f32),
        scratch_types=[
            pltpu.VMEM((kb,), jnp.int32),
            pltpu.VMEM((kb,), jnp.int32),
            pltpu.VMEM((kb, D), _f32),
            pltpu.VMEM((kb, 2 * D), _f32),
            pltpu.VMEM((kb,), jnp.int32),
            pltpu.VMEM((kb,), jnp.int32),
            pltpu.VMEM((kb, D), _f32),
            pltpu.VMEM((kb, 2 * D), _f32),
            pltpu.SemaphoreType.DMA,
            pltpu.SemaphoreType.DMA,
            pltpu.SemaphoreType.DMA,
            pltpu.SemaphoreType.DMA,
        ],
    )(_make_gather_body(eh // (NC * NS), kb))


_gather_kernel_1 = _make_gather_kernel(E1, 160)
_gather_kernel_2 = _make_gather_kernel(E2, KB)


def _stage_c_body(g_ref, wp1p_ref, bp1_ref, wp2_ref, bp2_ref,
                  wa1_ref, ba1_ref, wa2_ref, ba2_ref, cat_ref):
    g = g_ref[...]
    u0 = g[:, 0:H]
    dp = g[:, H:2 * H]
    vv = g[:, D:2 * D]
    t = jnp.maximum(jnp.dot(dp, wp1p_ref[...].T, preferred_element_type=_f32)
                    + bp1_ref[...], 0.0)
    delta = jnp.maximum(jnp.dot(t, wp2_ref[...].T, preferred_element_type=_f32)
                        + bp2_ref[...], 0.0)
    u = jnp.maximum(u0 + jnp.dot(delta, wa1_ref[...].T,
                                 preferred_element_type=_f32)
                    + ba1_ref[...], 0.0)
    alpha = jnp.maximum(jnp.dot(u, wa2_ref[...].T, preferred_element_type=_f32)
                        + ba2_ref[...], 0.0)
    ex = jnp.exp(alpha)
    exm = ex * (vv + delta)
    cat_ref[0] = jnp.concatenate([exm[:, 0:H], ex[:, 0:H]], axis=1)
    cat_ref[1] = jnp.concatenate([exm[:, H:D], ex[:, H:D]], axis=1)


def _stage_c(g, Wp1p, bp1_2, Wp2, bp2_2, Wa1, ba1_2, Wa2, ba2_2):
    eh = g.shape[0]
    nblk = eh // B_E
    wHH = pl.BlockSpec((H, H), lambda i: (0, 0))
    wDH = pl.BlockSpec((D, H), lambda i: (0, 0))
    wHD = pl.BlockSpec((H, D), lambda i: (0, 0))
    bH = pl.BlockSpec((1, H), lambda i: (0, 0))
    bD = pl.BlockSpec((1, D), lambda i: (0, 0))
    return pl.pallas_call(
        _stage_c_body,
        grid=(nblk,),
        in_specs=[
            pl.BlockSpec((B_E, 2 * D), lambda i: (i, 0)),
            wHH, bH, wDH, bD, wHD, bH, wDH, bD,
        ],
        out_specs=pl.BlockSpec((2, B_E, D), lambda i: (0, i, 0)),
        out_shape=jax.ShapeDtypeStruct((2, eh, D), _f32),
    )(g, Wp1p, bp1_2, Wp2, bp2_2, Wa1, ba1_2, Wa2, ba2_2)


def _make_scatter_body(ept):
    nchunks_s = ept // KD

    def _scatter_body(cat_h, dst_h, zeros_h, acc_out, idxd0, catb0, idxd1,
                      catb1, acc, seml0, seml1):
        c = lax.axis_index("c")
        s = lax.axis_index("s")
        r0 = s * RPT

        @pl.when(s < NS - 1)
        def _():
            pltpu.sync_copy(zeros_h.at[pl.ds(r0, RPT)],
                            acc.at[pl.ds(r0, RPT)])

        @pl.when(s == NS - 1)
        def _():
            pltpu.sync_copy(zeros_h.at[pl.ds(r0, RPT_LAST)],
                            acc.at[pl.ds(r0, RPT_LAST)])

        plsc.subcore_barrier()

        nchunks = nchunks_s
        slots = ((idxd0, catb0, seml0), (idxd1, catb1, seml1))

        def issue_load(slot, ci):
            idxd, catb, seml = slots[slot]
            e0 = s * ept + ci * KD
            pltpu.sync_copy(dst_h.at[pl.ds(e0, KD)], idxd)
            pltpu.async_copy(cat_h.at[c, pl.ds(e0, KD)], catb, seml)

        def wait_load(slot):
            idxd, catb, seml = slots[slot]
            pltpu.make_async_copy(cat_h.at[c, pl.ds(0, KD)], catb,
                                  seml).wait()

        def scatter(slot):
            idxd, catb, _ = slots[slot]
            pltpu.sync_copy(catb, acc.at[idxd], add=True)

        issue_load(0, 0)

        def pair(i, carry):
            c0 = 2 * i
            issue_load(1, c0 + 1)
            wait_load(0)
            scatter(0)

            @pl.when(c0 + 2 < nchunks)
            def _():
                issue_load(0, c0 + 2)

            wait_load(1)
            scatter(1)
            return carry

        lax.fori_loop(0, nchunks // 2, pair, 0)
        if nchunks % 2 == 1:
            wait_load(0)
            scatter(0)
        plsc.subcore_barrier()

        @pl.when(s < NS - 1)
        def _():
            pltpu.sync_copy(acc.at[pl.ds(r0, RPT)],
                            acc_out.at[c, pl.ds(r0, RPT)])

        @pl.when(s == NS - 1)
        def _():
            pltpu.sync_copy(acc.at[pl.ds(r0, RPT_LAST)],
                            acc_out.at[c, pl.ds(r0, RPT_LAST)])

    return _scatter_body


def _make_scatter_kernel(eh):
    return functools.partial(
        pl.kernel,
        mesh=plsc.VectorSubcoreMesh(core_axis_name="c", subcore_axis_name="s"),
        out_type=jax.ShapeDtypeStruct((NC, N, D), _f32),
        scratch_types=[
            pltpu.VMEM((KD,), jnp.int32),
            pltpu.VMEM((KD, D), _f32),
            pltpu.VMEM((KD,), jnp.int32),
            pltpu.VMEM((KD, D), _f32),
            pltpu.VMEM_SHARED((N, D), _f32),
            pltpu.SemaphoreType.DMA,
            pltpu.SemaphoreType.DMA,
        ],
    )(_make_scatter_body(eh // NS))


_scatter_kernel_1 = _make_scatter_kernel(E1)
_scatter_kernel_2 = _make_scatter_kernel(E2)


def _stage_e_body(al_ref, ar_ref, bl_ref, br_ref, wout_ref, bout_ref, o_ref):
    al = al_ref[0] + bl_ref[0]
    ar = ar_ref[0] + br_ref[0]
    rl = al[:, 0:H] / (al[:, H:D] + 1e-16)
    rr = ar[:, 0:H] / (ar[:, H:D] + 1e-16)
    w = wout_ref[...]
    o = (jnp.dot(rl, w[:, 0:H].T, preferred_element_type=_f32)
         + jnp.dot(rr, w[:, H:D].T, preferred_element_type=_f32)
         + bout_ref[...])
    o_ref[...] = jnp.maximum(o, 0.0)


def _stage_e(acc_a, acc_b, W_out, b_out2):
    nblk = N // B_N
    left = pl.BlockSpec((1, B_N, D), lambda i: (0, i, 0))
    right = pl.BlockSpec((1, B_N, D), lambda i: (1, i, 0))
    return pl.pallas_call(
        _stage_e_body,
        grid=(nblk,),
        in_specs=[
            left, right, left, right,
            pl.BlockSpec((D, D), lambda i: (0, 0)),
            pl.BlockSpec((1, D), lambda i: (0, 0)),
        ],
        out_specs=pl.BlockSpec((B_N, D), lambda i: (i, 0)),
        out_shape=jax.ShapeDtypeStruct((N, D), _f32),
    )(acc_a, acc_a, acc_b, acc_b, W_out, b_out2)


def kernel(x, pos, edge_index, W_in, b_in, W_lin, W_src, W_dst, Wp1, bp1,
           Wp2, bp2, Wa1, ba1, Wa2, ba2, W_out, b_out):
    src = edge_index[0]
    dst = edge_index[1]
    Wp1p = jnp.concatenate([Wp1, jnp.zeros((H, H - 3), _f32)], axis=1)
    b_in2 = b_in.reshape(1, D)
    bp1_2 = bp1.reshape(1, H)
    bp2_2 = bp2.reshape(1, D)
    ba1_2 = ba1.reshape(1, H)
    ba2_2 = ba2.reshape(1, D)
    b_out2 = b_out.reshape(1, D)

    drec, srec = _stage_a(x, pos, W_in, b_in2, W_lin, W_src, W_dst, Wa1)
    src1, src2 = src[:E1], src[E1:]
    dst1, dst2 = dst[:E1], dst[E1:]
    zeros_n = jnp.zeros((N, D), _f32)
    g1 = _gather_kernel_1(drec, srec, src1, dst1)
    g2 = _gather_kernel_2(drec, srec, src2, dst2)
    cat1 = _stage_c(g1, Wp1p, bp1_2, Wp2, bp2_2, Wa1, ba1_2, Wa2, ba2_2)
    cat2 = _stage_c(g2, Wp1p, bp1_2, Wp2, bp2_2, Wa1, ba1_2, Wa2, ba2_2)
    acc_a = _scatter_kernel_1(cat1, dst1, zeros_n)
    acc_b = _scatter_kernel_2(cat2, dst2, zeros_n)
    return _stage_e(acc_a, acc_b, W_out, b_out2)
